# baseline (speedup 1.0000x reference)
import functools
import math

import jax
import jax.numpy as jnp
from jax import lax
from jax.experimental import pallas as pl
from jax.experimental.pallas import tpu as pltpu
from jax.experimental.pallas import tpu_sc as plsc

B, CH, H, W = 16, 85, 160, 160
N, NC = 32, 80
HW = H * W
CHW = CH * HW
NPTS = B * N
NWORKERS = 32
PPT = NPTS // NWORKERS


def _sc_gather(x_flat, cy_flat, cx_flat):
    mesh = plsc.VectorSubcoreMesh(core_axis_name="c", subcore_axis_name="s")

    @functools.partial(
        pl.kernel,
        mesh=mesh,
        out_type=jax.ShapeDtypeStruct((NWORKERS, CH, PPT), jnp.float32),
        scratch_types=[
            pltpu.VMEM((PPT,), jnp.int32),
            pltpu.VMEM((PPT,), jnp.int32),
            pltpu.VMEM((CH * PPT,), jnp.int32),
            pltpu.VMEM((CH * PPT,), jnp.float32),
            pltpu.VMEM((CH, PPT), jnp.float32),
            pltpu.SemaphoreType.DMA,
        ],
    )
    def k(x_hbm, cy_hbm, cx_hbm, out_hbm, cyv, cxv, idxv, gv, gv2, sem):
        wid = lax.axis_index("s") * 2 + lax.axis_index("c")
        base_pt = wid * PPT
        pltpu.sync_copy(cy_hbm.at[pl.ds(base_pt, PPT)], cyv)
        pltpu.sync_copy(cx_hbm.at[pl.ds(base_pt, PPT)], cxv)
        base_idx = (wid // 2) * CHW + cyv[...] * W + cxv[...]

        def fill_c(c, carry):
            idxv[pl.ds(c * PPT, PPT)] = base_idx + c * HW
            return carry

        lax.fori_loop(0, CH, fill_c, 0)
        pltpu.async_copy(x_hbm.at[idxv], gv, sem).wait()

        def pack_c(c, carry):
            gv2[c, :] = gv[pl.ds(c * PPT, PPT)]
            return carry

        lax.fori_loop(0, CH, pack_c, 0)
        pltpu.sync_copy(gv2, out_hbm.at[wid])

    return k(x_flat, cy_flat, cx_flat)


def _atan(x):
    a = jnp.abs(x)
    inv = a > 1.0
    t = jnp.where(inv, 1.0 / jnp.maximum(a, 1e-30), a)
    s = t * t
    p = t * (0.99997726 + s * (-0.33262347 + s * (0.19354346 + s * (
        -0.11643287 + s * (0.05265332 + s * (-0.01172120))))))
    p = jnp.where(inv, (math.pi / 2.0) - p, p)
    return jnp.where(x < 0.0, -p, p)


def _softplus(x):
    return jnp.maximum(x, 0.0) + jnp.log(1.0 + jnp.exp(-jnp.abs(x)))


def _tc_body(x_ref, g_ref, gtb_ref, gtc_ref, cyc_ref, cxc_ref, cyr_ref,
             cxr_ref, out_ref):
    eps = 1e-10
    conf = x_ref[:, 0, :, :]
    neg_all = jnp.sum(_softplus(conf))

    bidx_c = lax.broadcasted_iota(jnp.int32, (1, NPTS), 1) // N
    bidx_r = lax.broadcasted_iota(jnp.int32, (NPTS, 1), 0) // N
    key_c = (bidx_c * H + cyc_ref[...]) * W + cxc_ref[...]
    key_r = (bidx_r * H + cyr_ref[...]) * W + cxr_ref[...]
    rr = lax.broadcasted_iota(jnp.int32, (NPTS, NPTS), 0)
    cc = lax.broadcasted_iota(jnp.int32, (NPTS, NPTS), 1)
    dup = jnp.any((key_r == key_c) & (rr < cc), axis=0, keepdims=True)
    uniq = jnp.where(dup, 0.0, 1.0)

    confc = g_ref[0:1, :]
    sp_neg = _softplus(confc)
    sp_pos = sp_neg - confc
    pos_cnt = jnp.sum(uniq)
    conf_pos = jnp.sum(uniq * sp_pos) / jnp.maximum(pos_cnt, 1.0)
    conf_neg = (neg_all - jnp.sum(uniq * sp_neg)) / jnp.maximum(
        float(B * H * W) - pos_cnt, 1.0)

    l1, t1 = g_ref[1:2, :], g_ref[2:3, :]
    r1, b1 = g_ref[3:4, :], g_ref[4:5, :]
    l2, t2 = gtb_ref[0:1, :], gtb_ref[1:2, :]
    r2, b2 = gtb_ref[2:3, :], gtb_ref[3:4, :]
    w1, h1 = r1 - l1, b1 - t1
    w2, h2 = r2 - l2, b2 - t2
    inter = jnp.clip(jnp.minimum(r1, r2) - jnp.maximum(l1, l2), 0.0, None) * \
            jnp.clip(jnp.minimum(b1, b2) - jnp.maximum(t1, t2), 0.0, None)
    union = w1 * h1 + w2 * h2 - inter + eps
    iou = inter / union
    cw = jnp.maximum(r1, r2) - jnp.minimum(l1, l2)
    chh = jnp.maximum(b1, b2) - jnp.minimum(t1, t2)
    c2 = cw ** 2 + chh ** 2 + eps
    rho2 = ((l2 + r2 - l1 - r1) ** 2 + (b2 + t2 - b1 - t1) ** 2) / 4.0
    v = 4.0 / (math.pi ** 2) * (_atan(w2 / (h2 + eps)) - _atan(w1 / (h1 + eps))) ** 2
    alpha = v / (v - iou + (1.0 + eps))
    ciou = iou - (rho2 / c2 + v * alpha)
    bbox_loss = -jnp.sum(ciou) / float(NPTS)

    cls_loss = -jnp.sum(g_ref[5:, :] * gtc_ref[...]) / float(NPTS)

    total = conf_pos + conf_neg + bbox_loss + cls_loss
    out_ref[...] = jnp.reshape(total, (1, 1))


def _tc_loss(x, g, gtb_t, gtc_t, cy_c, cx_c, cy_r, cx_r, interpret=False):
    return pl.pallas_call(
        _tc_body,
        grid=(1,),
        in_specs=[
            pl.BlockSpec((B, 1, H, W), lambda i: (0, 0, 0, 0)),
            pl.BlockSpec((CH, NPTS), lambda i: (0, 0)),
            pl.BlockSpec((4, NPTS), lambda i: (0, 0)),
            pl.BlockSpec((NC, NPTS), lambda i: (0, 0)),
            pl.BlockSpec((1, NPTS), lambda i: (0, 0)),
            pl.BlockSpec((1, NPTS), lambda i: (0, 0)),
            pl.BlockSpec((NPTS, 1), lambda i: (0, 0)),
            pl.BlockSpec((NPTS, 1), lambda i: (0, 0)),
        ],
        out_specs=pl.BlockSpec((1, 1), lambda i: (0, 0)),
        out_shape=jax.ShapeDtypeStruct((1, 1), jnp.float32),
        interpret=interpret,
    )(x, g, gtb_t, gtc_t, cy_c, cx_c, cy_r, cx_r)


def kernel(x, gt_bbox, gt_cls, cy, cx):
    g3 = _sc_gather(x.reshape(-1), cy.reshape(-1), cx.reshape(-1))
    g = jnp.transpose(g3, (1, 0, 2)).reshape(CH, NPTS)
    out = _tc_loss(
        x, g,
        gt_bbox.reshape(NPTS, 4).T,
        gt_cls.reshape(NPTS, NC).T,
        cy.reshape(1, NPTS), cx.reshape(1, NPTS),
        cy.reshape(NPTS, 1), cx.reshape(NPTS, 1),
    )
    return out[0, 0]
